# baseline (speedup 1.0000x reference)
import functools

import jax
import jax.numpy as jnp
from jax import lax
from jax.experimental import pallas as pl
from jax.experimental.pallas import tpu as pltpu
from jax.experimental.pallas import tpu_sc as plsc

N = 10000
NPAD = 10240
DIN = 128
DH = 64
DOUT = 32
E = 320000

NC = 2
NS = 16
NW = NC * NS
W = 128
NWIN = E // W
XTRA = NWIN - 78 * NW
CH = 79
ROWS_PER_SUB = NPAD // NS

_mesh = plsc.VectorSubcoreMesh(
    core_axis_name="c", subcore_axis_name="s", num_cores=NC, num_subcores=NS
)


def _worker_windows(wid):
    w0 = wid * 78 + jnp.minimum(wid, XTRA)
    chw = 78 + jnp.where(wid < XTRA, 1, 0)
    return w0, chw


def _load_windows(eidx, plane, w0, idx_v, wid):
    @pl.when(wid < XTRA)
    def _():
        pltpu.sync_copy(eidx.at[plane, pl.ds(w0, 79)], idx_v)

    @pl.when(wid >= XTRA)
    def _():
        pltpu.sync_copy(eidx.at[plane, pl.ds(w0, 78)], idx_v.at[pl.ds(0, 78)])


@functools.partial(
    pl.kernel,
    out_type=jax.ShapeDtypeStruct((NC, NPAD // 128, 128), jnp.float32),
    mesh=_mesh,
    scratch_types=[
        pltpu.VMEM((CH, W), jnp.int32),
        pltpu.VMEM((NPAD // 2,), jnp.float32),
        pltpu.VMEM((NPAD // 256, 128), jnp.float32),
        pltpu.VMEM((W,), jnp.float32),
        pltpu.VMEM_SHARED((NPAD,), jnp.float32),
        pltpu.SemaphoreType.DMA,
    ],
    compiler_params=pltpu.CompilerParams(use_tc_tiling_on_sc=False),
)
def _deg_kernel(eidx_hbm, deg_out, idx_v, buf_v, buf2_v, ones_v, hist_sh, dsem):
    c = lax.axis_index("c")
    s = lax.axis_index("s")
    wid = c * NS + s
    base = s * ROWS_PER_SUB
    w0, chw = _worker_windows(wid)

    for i in range(ROWS_PER_SUB // 16):
        buf_v[pl.ds(i * 16, 16)] = jnp.zeros((16,), jnp.float32)
    for i in range(W // 16):
        ones_v[pl.ds(i * 16, 16)] = jnp.full((16,), 1.0, jnp.float32)

    pltpu.sync_copy(buf_v.at[pl.ds(0, ROWS_PER_SUB)], hist_sh.at[pl.ds(base, ROWS_PER_SUB)])
    plsc.subcore_barrier()

    _load_windows(eidx_hbm, 0, w0, idx_v, wid)

    def win(j, carry):
        @pl.when(j < chw)
        def _():
            pltpu.async_copy(ones_v, hist_sh.at[idx_v.at[j]], dsem, add=True)

        return carry

    lax.fori_loop(0, CH, win, 0)

    def drain(j, carry):
        @pl.when(j < chw)
        def _():
            pltpu.make_async_copy(ones_v, hist_sh.at[idx_v.at[j]], dsem).wait()

        return carry

    lax.fori_loop(0, CH, drain, 0)
    plsc.subcore_barrier()

    @pl.when(s < 2)
    def _():
        half = NPAD // 2
        pltpu.sync_copy(hist_sh.at[pl.ds(s * half, half)], buf_v)

        def repack(i, carry):
            for k in range(128 // 16):
                buf2_v[i, pl.ds(k * 16, 16)] = buf_v[pl.ds(i * 128 + k * 16, 16)]
            return carry

        lax.fori_loop(0, NPAD // 256, repack, 0)
        pltpu.sync_copy(buf2_v, deg_out.at[c, pl.ds(s * (NPAD // 256), NPAD // 256)])


NB = 2


def _make_spmm(D, GK, staged):
    NG = -(-CH // GK)
    scratch = [
        pltpu.VMEM((CH, W), jnp.int32),
        pltpu.VMEM((CH, W), jnp.int32),
        pltpu.VMEM((NB, GK, W, D), jnp.float32),
        pltpu.VMEM_SHARED((NPAD, D), jnp.float32),
        [[pltpu.SemaphoreType.DMA for _ in range(GK)] for _ in range(NB)],
        [pltpu.SemaphoreType.DMA for _ in range(NB)],
    ]
    if staged:
        scratch.append(pltpu.VMEM_SHARED((NPAD, D), jnp.float32))

    @functools.partial(
        pl.kernel,
        out_type=jax.ShapeDtypeStruct((NC, NPAD, D), jnp.float32),
        mesh=_mesh,
        scratch_types=scratch,
        compiler_params=pltpu.CompilerParams(use_tc_tiling_on_sc=False),
    )
    def spmm_k(xp_hbm, eidx_hbm, acc_out, idxc_v, idxr_v, rows_v, acc_sh, gsems, ssems, *rest):
        c = lax.axis_index("c")
        s = lax.axis_index("s")
        wid = c * NS + s
        base = s * ROWS_PER_SUB
        w0, chw = _worker_windows(wid)

        table = rest[0] if staged else xp_hbm
        if staged:
            pltpu.sync_copy(xp_hbm.at[pl.ds(base, ROWS_PER_SUB)], rest[0].at[pl.ds(base, ROWS_PER_SUB)])

        def zrow(i, carry):
            for j in range(D // 16):
                rows_v[0, 0, i, pl.ds(j * 16, 16)] = jnp.zeros((16,), jnp.float32)
            return carry

        lax.fori_loop(0, W, zrow, 0)
        for k in range(ROWS_PER_SUB // W):
            pltpu.sync_copy(rows_v.at[0, 0], acc_sh.at[pl.ds(base + k * W, W)])
        plsc.subcore_barrier()

        _load_windows(eidx_hbm, 1, w0, idxc_v, wid)
        _load_windows(eidx_hbm, 0, w0, idxr_v, wid)

        def issue_gathers(g, bank):
            for b in range(GK):
                j = g * GK + b

                @pl.when(j < chw)
                def _():
                    pltpu.async_copy(
                        table.at[idxc_v.at[j]], rows_v.at[bank, b], gsems[bank][b]
                    )

        def process_group(g, bank):
            for b in range(GK):
                j = g * GK + b

                @pl.when(j < chw)
                def _():
                    pltpu.make_async_copy(
                        table.at[idxc_v.at[j]], rows_v.at[bank, b], gsems[bank][b]
                    ).wait()
                    pltpu.async_copy(
                        rows_v.at[bank, b], acc_sh.at[idxr_v.at[j]], ssems[bank], add=True
                    )

            for b in range(GK):
                j = g * GK + b

                @pl.when(j < chw)
                def _():
                    pltpu.make_async_copy(
                        rows_v.at[bank, b], acc_sh.at[idxr_v.at[j]], ssems[bank]
                    ).wait()

            issue_gathers(g + 2, bank)

        issue_gathers(0, 0)
        issue_gathers(1, 1)

        def outer(i, carry):
            process_group(2 * i, 0)
            process_group(2 * i + 1, 1)
            return carry

        lax.fori_loop(0, NG // 2, outer, 0)
        plsc.subcore_barrier()

        for k in range(ROWS_PER_SUB // W):
            pltpu.sync_copy(acc_sh.at[pl.ds(base + k * W, W)], rows_v.at[0, 0])
            pltpu.sync_copy(rows_v.at[0, 0], acc_out.at[c, pl.ds(base + k * W, W)])

    return spmm_k


_spmm64 = _make_spmm(DH, 4, False)
_spmm32 = _make_spmm(DOUT, 8, True)


RB = 2048
_GRID = (NPAD // RB,)
_DB = RB // 128


def _dis_block(degp_ref):
    i = pl.program_id(0)
    deg = degp_ref[0] + degp_ref[1] + 1.0
    r = lax.broadcasted_iota(jnp.int32, (_DB, 128), 0)
    l = lax.broadcasted_iota(jnp.int32, (_DB, 128), 1)
    valid = (i * RB + r * 128 + l) < N
    dis8 = jnp.where(valid, lax.rsqrt(deg), 0.0)

    rr = lax.broadcasted_iota(jnp.int32, (RB, _DB), 0)
    ii = lax.broadcasted_iota(jnp.int32, (RB, _DB), 1)
    sel_row = jnp.where(rr // 128 == ii, 1.0, 0.0)
    t = lax.dot_general(
        sel_row, dis8, (((1,), (0,)), ((), ())), preferred_element_type=jnp.float32
    )
    r2 = lax.broadcasted_iota(jnp.int32, (RB, 128), 0)
    l2 = lax.broadcasted_iota(jnp.int32, (RB, 128), 1)
    t = jnp.where(l2 == r2 % 128, t, 0.0)
    return lax.dot_general(
        t,
        jnp.ones((128, 1), jnp.float32),
        (((1,), (0,)), ((), ())),
        preferred_element_type=jnp.float32,
    )


_DEGP_SPEC = pl.BlockSpec((NC, _DB, 128), lambda i: (0, i, 0))


def _stage1_body(x_ref, w1_ref, degp_ref, x1p_ref):
    i = pl.program_id(0)
    r = lax.broadcasted_iota(jnp.int32, (RB, DIN), 0)
    x = jnp.where(i * RB + r < N, x_ref[...], 0.0)
    y = lax.dot_general(
        x, w1_ref[...], (((1,), (1,)), ((), ())), preferred_element_type=jnp.float32
    )
    x1p_ref[...] = _dis_block(degp_ref) * y


_stage1_call = pl.pallas_call(
    _stage1_body,
    grid=_GRID,
    in_specs=[
        pl.BlockSpec((RB, DIN), lambda i: (i, 0)),
        pl.BlockSpec((DH, DIN), lambda i: (0, 0)),
        _DEGP_SPEC,
    ],
    out_specs=pl.BlockSpec((RB, DH), lambda i: (i, 0)),
    out_shape=jax.ShapeDtypeStruct((NPAD, DH), jnp.float32),
)


def _stage2_body(acc_ref, x1p_ref, degp_ref, b1_ref, w2_ref, x2p_ref):
    dis = _dis_block(degp_ref)
    h = dis * (acc_ref[0] + acc_ref[1] + x1p_ref[...]) + b1_ref[...]
    h = jnp.maximum(h, 0.0)
    y2 = lax.dot_general(
        h, w2_ref[...], (((1,), (1,)), ((), ())), preferred_element_type=jnp.float32
    )
    x2p_ref[...] = dis * y2


_stage2_call = pl.pallas_call(
    _stage2_body,
    grid=_GRID,
    in_specs=[
        pl.BlockSpec((NC, RB, DH), lambda i: (0, i, 0)),
        pl.BlockSpec((RB, DH), lambda i: (i, 0)),
        _DEGP_SPEC,
        pl.BlockSpec((1, DH), lambda i: (0, 0)),
        pl.BlockSpec((DOUT, DH), lambda i: (0, 0)),
    ],
    out_specs=pl.BlockSpec((RB, DOUT), lambda i: (i, 0)),
    out_shape=jax.ShapeDtypeStruct((NPAD, DOUT), jnp.float32),
)


def _stage3_body(acc_ref, x2p_ref, degp_ref, b2_ref, out_ref):
    out_ref[...] = (
        _dis_block(degp_ref) * (acc_ref[0] + acc_ref[1] + x2p_ref[...]) + b2_ref[...]
    )


_stage3_call = pl.pallas_call(
    _stage3_body,
    grid=_GRID,
    in_specs=[
        pl.BlockSpec((NC, RB, DOUT), lambda i: (0, i, 0)),
        pl.BlockSpec((RB, DOUT), lambda i: (i, 0)),
        _DEGP_SPEC,
        pl.BlockSpec((1, DOUT), lambda i: (0, 0)),
    ],
    out_specs=pl.BlockSpec((RB, DOUT), lambda i: (i, 0)),
    out_shape=jax.ShapeDtypeStruct((N, DOUT), jnp.float32),
)


def kernel(edge_index, node_features, W1, b1, W2, b2):
    eidx = edge_index.astype(jnp.int32).reshape(2, NWIN, W)

    degp = _deg_kernel(eidx)
    x1p = _stage1_call(node_features, W1, degp)
    acc1 = _spmm64(x1p, eidx)
    x2p = _stage2_call(acc1, x1p, degp, b1.reshape(1, DH), W2)
    acc2 = _spmm32(x2p, eidx)
    return _stage3_call(acc2, x2p, degp, b2.reshape(1, DOUT))


# speedup vs baseline: 1.2398x; 1.2398x over previous
import functools

import jax
import jax.numpy as jnp
from jax import lax
from jax.experimental import pallas as pl
from jax.experimental.pallas import tpu as pltpu
from jax.experimental.pallas import tpu_sc as plsc

N = 10000
NPAD = 10240
DIN = 128
DH = 64
DOUT = 32
E = 320000

NC = 2
NS = 16
NW = NC * NS
W = 128
NWIN = E // W
XTRA = NWIN - 78 * NW
CH = 79
ROWS_PER_SUB = NPAD // NS

_mesh = plsc.VectorSubcoreMesh(
    core_axis_name="c", subcore_axis_name="s", num_cores=NC, num_subcores=NS
)


def _worker_windows(wid):
    w0 = wid * 78 + jnp.minimum(wid, XTRA)
    chw = 78 + jnp.where(wid < XTRA, 1, 0)
    return w0, chw


def _load_windows(eidx, plane, w0, idx_v, wid):
    @pl.when(wid < XTRA)
    def _():
        pltpu.sync_copy(eidx.at[plane, pl.ds(w0, 79)], idx_v)

    @pl.when(wid >= XTRA)
    def _():
        pltpu.sync_copy(eidx.at[plane, pl.ds(w0, 78)], idx_v.at[pl.ds(0, 78)])


@functools.partial(
    pl.kernel,
    out_type=jax.ShapeDtypeStruct((NC, NPAD // 128, 128), jnp.float32),
    mesh=_mesh,
    scratch_types=[
        pltpu.VMEM((CH, W), jnp.int32),
        pltpu.VMEM((NPAD // 2,), jnp.float32),
        pltpu.VMEM((NPAD // 256, 128), jnp.float32),
        pltpu.VMEM((W,), jnp.float32),
        pltpu.VMEM_SHARED((NPAD,), jnp.float32),
        pltpu.SemaphoreType.DMA,
    ],
    compiler_params=pltpu.CompilerParams(use_tc_tiling_on_sc=False),
)
def _deg_kernel(eidx_hbm, deg_out, idx_v, buf_v, buf2_v, ones_v, hist_sh, dsem):
    c = lax.axis_index("c")
    s = lax.axis_index("s")
    wid = c * NS + s
    base = s * ROWS_PER_SUB
    w0, chw = _worker_windows(wid)

    for i in range(ROWS_PER_SUB // 16):
        buf_v[pl.ds(i * 16, 16)] = jnp.zeros((16,), jnp.float32)
    for i in range(W // 16):
        ones_v[pl.ds(i * 16, 16)] = jnp.full((16,), 1.0, jnp.float32)

    pltpu.sync_copy(buf_v.at[pl.ds(0, ROWS_PER_SUB)], hist_sh.at[pl.ds(base, ROWS_PER_SUB)])
    plsc.subcore_barrier()

    _load_windows(eidx_hbm, 0, w0, idx_v, wid)

    def win(j, carry):
        @pl.when(j < chw)
        def _():
            pltpu.async_copy(ones_v, hist_sh.at[idx_v.at[j]], dsem, add=True)

        return carry

    lax.fori_loop(0, CH, win, 0)

    def drain(j, carry):
        @pl.when(j < chw)
        def _():
            pltpu.make_async_copy(ones_v, hist_sh.at[idx_v.at[j]], dsem).wait()

        return carry

    lax.fori_loop(0, CH, drain, 0)
    plsc.subcore_barrier()

    @pl.when(s < 2)
    def _():
        half = NPAD // 2
        pltpu.sync_copy(hist_sh.at[pl.ds(s * half, half)], buf_v)

        def repack(i, carry):
            for k in range(128 // 16):
                buf2_v[i, pl.ds(k * 16, 16)] = buf_v[pl.ds(i * 128 + k * 16, 16)]
            return carry

        lax.fori_loop(0, NPAD // 256, repack, 0)
        pltpu.sync_copy(buf2_v, deg_out.at[c, pl.ds(s * (NPAD // 256), NPAD // 256)])


NB = 2


def _make_spmm(D, GK, staged):
    NG = -(-CH // GK)
    scratch = [
        pltpu.VMEM((CH, W), jnp.int32),
        pltpu.VMEM((CH, W), jnp.int32),
        pltpu.VMEM((NB, GK, W, D), jnp.float32),
        pltpu.VMEM_SHARED((NPAD, D), jnp.float32),
        [[pltpu.SemaphoreType.DMA for _ in range(GK)] for _ in range(NB)],
        [pltpu.SemaphoreType.DMA for _ in range(NB)],
    ]
    if staged:
        scratch.append(pltpu.VMEM_SHARED((NPAD, D), jnp.float32))

    @functools.partial(
        pl.kernel,
        out_type=jax.ShapeDtypeStruct((NC, NPAD, D), jnp.float32),
        mesh=_mesh,
        scratch_types=scratch,
        compiler_params=pltpu.CompilerParams(use_tc_tiling_on_sc=False),
    )
    def spmm_k(xp_hbm, eidx_hbm, acc_out, idxc_v, idxr_v, rows_v, acc_sh, gsems, ssems, *rest):
        c = lax.axis_index("c")
        s = lax.axis_index("s")
        wid = c * NS + s
        base = s * ROWS_PER_SUB
        w0, chw = _worker_windows(wid)

        table = rest[0] if staged else xp_hbm
        if staged:
            pltpu.sync_copy(xp_hbm.at[pl.ds(base, ROWS_PER_SUB)], rest[0].at[pl.ds(base, ROWS_PER_SUB)])

        def zrow(i, carry):
            for j in range(D // 16):
                rows_v[0, 0, i, pl.ds(j * 16, 16)] = jnp.zeros((16,), jnp.float32)
            return carry

        lax.fori_loop(0, W, zrow, 0)
        for k in range(ROWS_PER_SUB // W):
            pltpu.sync_copy(rows_v.at[0, 0], acc_sh.at[pl.ds(base + k * W, W)])
        plsc.subcore_barrier()

        _load_windows(eidx_hbm, 1, w0, idxc_v, wid)
        _load_windows(eidx_hbm, 0, w0, idxr_v, wid)

        def issue_gathers(g, bank):
            for b in range(GK):
                j = g * GK + b

                @pl.when(j < chw)
                def _():
                    pltpu.async_copy(
                        table.at[idxc_v.at[j]], rows_v.at[bank, b], gsems[bank][b]
                    )

        def process_group(g, bank):
            for b in range(GK):
                j = g * GK + b

                @pl.when(j < chw)
                def _():
                    pltpu.make_async_copy(
                        table.at[idxc_v.at[j]], rows_v.at[bank, b], gsems[bank][b]
                    ).wait()
                    pltpu.async_copy(
                        rows_v.at[bank, b], acc_sh.at[idxr_v.at[j]], ssems[bank], add=True
                    )

            for b in range(GK):
                j = g * GK + b

                @pl.when(j < chw)
                def _():
                    pltpu.make_async_copy(
                        rows_v.at[bank, b], acc_sh.at[idxr_v.at[j]], ssems[bank]
                    ).wait()

            issue_gathers(g + 2, bank)

        issue_gathers(0, 0)
        issue_gathers(1, 1)

        def outer(i, carry):
            process_group(2 * i, 0)
            process_group(2 * i + 1, 1)
            return carry

        lax.fori_loop(0, NG // 2, outer, 0)
        plsc.subcore_barrier()

        for k in range(ROWS_PER_SUB // W):
            pltpu.sync_copy(acc_sh.at[pl.ds(base + k * W, W)], rows_v.at[0, 0])
            pltpu.sync_copy(rows_v.at[0, 0], acc_out.at[c, pl.ds(base + k * W, W)])

    return spmm_k


_spmm64 = _make_spmm(DH, 4, False)
_spmm32 = _make_spmm(DOUT, 8, False)


RB = 2048
_GRID = (NPAD // RB,)
_DB = RB // 128


def _dis8_block(degp_ref):
    i = pl.program_id(0)
    deg = degp_ref[0] + degp_ref[1] + 1.0
    r = lax.broadcasted_iota(jnp.int32, (_DB, 128), 0)
    l = lax.broadcasted_iota(jnp.int32, (_DB, 128), 1)
    valid = (i * RB + r * 128 + l) < N
    return jnp.where(valid, lax.rsqrt(deg), 0.0)


def _dis_col(dis8, m, t, X):
    rr = lax.broadcasted_iota(jnp.int32, (X, _DB), 0)
    ii = lax.broadcasted_iota(jnp.int32, (X, _DB), 1)
    sel = jnp.where((m * rr + t) // 128 == ii, 1.0, 0.0)
    tt = lax.dot_general(
        sel, dis8, (((1,), (0,)), ((), ())), preferred_element_type=jnp.float32
    )
    r2 = lax.broadcasted_iota(jnp.int32, (X, 128), 0)
    l2 = lax.broadcasted_iota(jnp.int32, (X, 128), 1)
    tt = jnp.where(l2 == (m * r2 + t) % 128, tt, 0.0)
    return lax.dot_general(
        tt,
        jnp.ones((128, 1), jnp.float32),
        (((1,), (0,)), ((), ())),
        preferred_element_type=jnp.float32,
    )


_DEGP_SPEC = pl.BlockSpec((NC, _DB, 128), lambda i: (0, i, 0))
RBP = RB // 2
RBQ = RB // 4
PR = NPAD // 2
QR = NPAD // 4


def _lane_select(cols, X, span):
    l = lax.broadcasted_iota(jnp.int32, (X, span * len(cols)), 1)
    out = jnp.zeros((X, span * len(cols)), jnp.float32)
    for t, col in enumerate(cols):
        out = jnp.where((l >= t * span) & (l < (t + 1) * span), col, out)
    return out


def _stage1_body(x_ref, w1_ref, degp_ref, x1p_ref):
    i = pl.program_id(0)
    r = lax.broadcasted_iota(jnp.int32, (RBP, 2 * DIN), 0)
    x = jnp.where(i * RBP + r < N // 2, x_ref[...], 0.0)
    y = lax.dot_general(
        x, w1_ref[...], (((1,), (0,)), ((), ())), preferred_element_type=jnp.float32
    )
    dis8 = _dis8_block(degp_ref)
    cols = [_dis_col(dis8, 2, t, RBP) for t in range(2)]
    x1p_ref[...] = _lane_select(cols, RBP, DH) * y


_stage1_call = pl.pallas_call(
    _stage1_body,
    grid=_GRID,
    in_specs=[
        pl.BlockSpec((RBP, 2 * DIN), lambda i: (i, 0)),
        pl.BlockSpec((2 * DIN, 2 * DH), lambda i: (0, 0)),
        _DEGP_SPEC,
    ],
    out_specs=pl.BlockSpec((RBP, 128), lambda i: (i, 0)),
    out_shape=jax.ShapeDtypeStruct((PR, 128), jnp.float32),
)


def _stage2_body(acc_ref, x1p_ref, degp_ref, b1_ref, w2_ref, x2p_ref):
    dis8 = _dis8_block(degp_ref)
    cols = [_dis_col(dis8, 2, t, RBP) for t in range(2)]
    dis_p = _lane_select(cols, RBP, DH)
    h = dis_p * (acc_ref[0] + acc_ref[1] + x1p_ref[...]) + b1_ref[...]
    h = jnp.maximum(h, 0.0)
    y2 = lax.dot_general(
        h, w2_ref[...], (((1,), (0,)), ((), ())), preferred_element_type=jnp.float32
    )
    x2p_ref[...] = _lane_select(cols, RBP, DOUT) * y2


_stage2_call = pl.pallas_call(
    _stage2_body,
    grid=_GRID,
    in_specs=[
        pl.BlockSpec((NC, RBP, 128), lambda i: (0, i, 0)),
        pl.BlockSpec((RBP, 128), lambda i: (i, 0)),
        _DEGP_SPEC,
        pl.BlockSpec((1, 2 * DH), lambda i: (0, 0)),
        pl.BlockSpec((2 * DH, 2 * DOUT), lambda i: (0, 0)),
    ],
    out_specs=pl.BlockSpec((RBP, 2 * DOUT), lambda i: (i, 0)),
    out_shape=jax.ShapeDtypeStruct((PR, 2 * DOUT), jnp.float32),
)


def _stage3_body(acc_ref, x2p_ref, degp_ref, b2_ref, out_ref):
    dis8 = _dis8_block(degp_ref)
    cols = [_dis_col(dis8, 4, t, RBQ) for t in range(4)]
    out_ref[...] = (
        _lane_select(cols, RBQ, DOUT) * (acc_ref[0] + acc_ref[1] + x2p_ref[...])
        + b2_ref[...]
    )


_stage3_call = pl.pallas_call(
    _stage3_body,
    grid=_GRID,
    in_specs=[
        pl.BlockSpec((NC, RBQ, 128), lambda i: (0, i, 0)),
        pl.BlockSpec((RBQ, 128), lambda i: (i, 0)),
        _DEGP_SPEC,
        pl.BlockSpec((1, 128), lambda i: (0, 0)),
    ],
    out_specs=pl.BlockSpec((RBQ, 128), lambda i: (i, 0)),
    out_shape=jax.ShapeDtypeStruct((N // 4, 128), jnp.float32),
)


def kernel(edge_index, node_features, W1, b1, W2, b2):
    eidx = edge_index.astype(jnp.int32).reshape(2, NWIN, W)

    w1big = (
        jnp.zeros((2 * DIN, 2 * DH), jnp.float32)
        .at[:DIN, :DH].set(W1.T)
        .at[DIN:, DH:].set(W1.T)
    )
    w2big = (
        jnp.zeros((2 * DH, 2 * DOUT), jnp.float32)
        .at[:DH, :DOUT].set(W2.T)
        .at[DH:, DOUT:].set(W2.T)
    )
    b1p = jnp.concatenate([b1, b1]).reshape(1, 2 * DH)
    b2q = jnp.concatenate([b2, b2, b2, b2]).reshape(1, 128)

    degp = _deg_kernel(eidx)
    x1p_pair = _stage1_call(node_features.reshape(N // 2, 2 * DIN), w1big, degp)
    x1p = x1p_pair.reshape(NPAD, DH)
    acc1 = _spmm64(x1p, eidx)
    x2p_pair = _stage2_call(
        acc1.reshape(NC, PR, 128), x1p_pair, degp, b1p, w2big
    )
    x2p = x2p_pair.reshape(NPAD, DOUT)
    acc2 = _spmm32(x2p, eidx)
    outq = _stage3_call(
        acc2.reshape(NC, QR, 128), x2p.reshape(QR, 128), degp, b2q
    )
    return outq.reshape(N, DOUT)
